# Initial kernel scaffold; baseline (speedup 1.0000x reference)
import jax
import jax.numpy as jnp
from jax.experimental import pallas as pl


def kernel(x_ncdhw, w_oidhw, gamma, beta):
    raise NotImplementedError("write your pallas kernel here")


# speedup vs baseline: 2.4563x; 2.4563x over previous
import jax
import jax.numpy as jnp
from jax import lax
from jax.experimental import pallas as pl
from jax.experimental.pallas import tpu as pltpu

_EPS = 1e-5
_NB = 2
_NB2 = 4
_PADF = 384


def _conv3d_bn_relu(x_ncdhw, w_oidhw, gamma, beta):
    N, Cin, D, H, W = x_ncdhw.shape
    Cout = w_oidhw.shape[0]
    HW = H * W
    S = D * HW
    K9 = 9 * Cin
    PADF = _PADF
    Lin = PADF + S + PADF
    Lc = S + 2 * HW

    x3 = x_ncdhw.reshape(N, Cin, S)

    w3 = jnp.transpose(w_oidhw, (2, 0, 3, 4, 1)).reshape(3, Cout, K9)
    w3 = w3.astype(jnp.bfloat16)

    q = jnp.arange(Lc, dtype=jnp.int32)
    h_i = (q % HW) // W
    w_i = q % W
    masks = []
    offs = []
    for kh in range(3):
        for kw in range(3):
            offs.append((kh - 1) * W + (kw - 1))
            ok = ((h_i + (kh - 1) >= 0) & (h_i + (kh - 1) < H)
                  & (w_i + (kw - 1) >= 0) & (w_i + (kw - 1) < W))
            masks.append(ok)
    offs = tuple(offs)
    mask_arr = jnp.stack(masks).astype(jnp.bfloat16)

    NB = _NB
    G = N // NB

    def conv_kernel(x_ref, w_ref, mask_ref, y_ref, ps_ref, pq_ref,
                    xs_ref, col_ref):
        for i in range(NB):
            xs_ref[i, :, :PADF] = jnp.zeros((Cin, PADF), jnp.bfloat16)
            xs_ref[i, :, PADF + S:] = jnp.zeros((Cin, Lin - PADF - S),
                                                jnp.bfloat16)
            xs_ref[i, :, PADF:PADF + S] = x_ref[i].astype(jnp.bfloat16)
        for j, off in enumerate(offs):
            start = PADF - HW + off
            m = mask_ref[j:j + 1, :]
            for i in range(NB):
                col_ref[i, j * Cin:(j + 1) * Cin, :] = (
                    xs_ref[i, :, start:start + Lc] * m)
        ps = None
        pq = None
        for i in range(NB):
            acc = (jnp.dot(w_ref[0], col_ref[i, :, 0:S],
                           preferred_element_type=jnp.float32)
                   + jnp.dot(w_ref[1], col_ref[i, :, HW:HW + S],
                             preferred_element_type=jnp.float32)
                   + jnp.dot(w_ref[2], col_ref[i, :, 2 * HW:2 * HW + S],
                             preferred_element_type=jnp.float32))
            y_ref[i] = acc.astype(jnp.bfloat16)
            s = jnp.sum(acc, axis=1, keepdims=True)
            t = jnp.sum(acc * acc, axis=1, keepdims=True)
            ps = s if ps is None else ps + s
            pq = t if pq is None else pq + t
        ps_ref[0] = ps
        pq_ref[0] = pq

    y, psum, psq = pl.pallas_call(
        conv_kernel,
        out_shape=(
            jax.ShapeDtypeStruct((N, Cout, S), jnp.bfloat16),
            jax.ShapeDtypeStruct((G, Cout, 1), jnp.float32),
            jax.ShapeDtypeStruct((G, Cout, 1), jnp.float32),
        ),
        grid_spec=pltpu.PrefetchScalarGridSpec(
            num_scalar_prefetch=0,
            grid=(G,),
            in_specs=[
                pl.BlockSpec((NB, Cin, S), lambda g: (g, 0, 0)),
                pl.BlockSpec((3, Cout, K9), lambda g: (0, 0, 0)),
                pl.BlockSpec((9, Lc), lambda g: (0, 0)),
            ],
            out_specs=[
                pl.BlockSpec((NB, Cout, S), lambda g: (g, 0, 0)),
                pl.BlockSpec((1, Cout, 1), lambda g: (g, 0, 0)),
                pl.BlockSpec((1, Cout, 1), lambda g: (g, 0, 0)),
            ],
            scratch_shapes=[
                pltpu.VMEM((NB, Cin, Lin), jnp.bfloat16),
                pltpu.VMEM((NB, K9, Lc), jnp.bfloat16),
            ],
        ),
        compiler_params=pltpu.CompilerParams(
            dimension_semantics=("parallel",),
            vmem_limit_bytes=64 * 1024 * 1024,
        ),
    )(x3, w3, mask_arr)

    count = N * S
    tot = jnp.sum(psum[:, :, 0], axis=0)
    tot_sq = jnp.sum(psq[:, :, 0], axis=0)
    mean = tot / count
    var = tot_sq / count - mean * mean
    inv = gamma * lax.rsqrt(var + _EPS)
    scale = inv.reshape(Cout, 1)
    shift = (beta - mean * inv).reshape(Cout, 1)

    NB2 = _NB2
    G2 = N // NB2

    def bn_relu_kernel(y_ref, sc_ref, sh_ref, o_ref):
        o_ref[...] = jnp.maximum(
            y_ref[...].astype(jnp.float32) * sc_ref[...] + sh_ref[...], 0.0)

    out = pl.pallas_call(
        bn_relu_kernel,
        out_shape=jax.ShapeDtypeStruct((N, Cout, S), jnp.float32),
        grid_spec=pltpu.PrefetchScalarGridSpec(
            num_scalar_prefetch=0,
            grid=(G2,),
            in_specs=[
                pl.BlockSpec((NB2, Cout, S), lambda g: (g, 0, 0)),
                pl.BlockSpec((Cout, 1), lambda g: (0, 0)),
                pl.BlockSpec((Cout, 1), lambda g: (0, 0)),
            ],
            out_specs=pl.BlockSpec((NB2, Cout, S), lambda g: (g, 0, 0)),
        ),
        compiler_params=pltpu.CompilerParams(
            dimension_semantics=("parallel",),
            vmem_limit_bytes=64 * 1024 * 1024,
        ),
    )(y, scale, shift)

    return out.reshape(N, Cout, D, H, W)


def kernel(x_ncdhw, w_oidhw, gamma, beta):
    return _conv3d_bn_relu(x_ncdhw, w_oidhw, gamma, beta)


# speedup vs baseline: 2.9394x; 1.1967x over previous
import jax
import jax.numpy as jnp
from jax import lax
from jax.experimental import pallas as pl
from jax.experimental.pallas import tpu as pltpu

_EPS = 1e-5
_NB = 8
_PADF = 384


def _conv3d_bn_relu(x_ncdhw, w_oidhw, gamma, beta):
    N, Cin, D, H, W = x_ncdhw.shape
    Cout = w_oidhw.shape[0]
    HW = H * W
    S = D * HW
    K9 = 9 * Cin
    PADF = _PADF
    Lin = PADF + S + PADF
    Lc = S + 2 * HW

    x3 = x_ncdhw.reshape(N, Cin, S)

    w3 = jnp.transpose(w_oidhw, (2, 0, 3, 4, 1)).reshape(3, Cout, K9)
    w3 = w3.astype(jnp.bfloat16)

    gb = jnp.stack([gamma, beta]).reshape(2, Cout, 1)

    q = jnp.arange(Lc, dtype=jnp.int32)
    h_i = (q % HW) // W
    w_i = q % W
    masks = []
    offs = []
    for kh in range(3):
        for kw in range(3):
            offs.append((kh - 1) * W + (kw - 1))
            ok = ((h_i + (kh - 1) >= 0) & (h_i + (kh - 1) < H)
                  & (w_i + (kw - 1) >= 0) & (w_i + (kw - 1) < W))
            masks.append(ok)
    offs = tuple(offs)
    mask_arr = jnp.stack(masks).astype(jnp.bfloat16)

    NB = min(_NB, N)
    G = N // NB
    count = N * S

    def fused_kernel(x_ref, w_ref, mask_ref, gb_ref, o_ref,
                     xs_ref, col_ref, y_ref, ss_ref, sq_ref, sc_ref):
        p = pl.program_id(0)
        g = pl.program_id(1)

        @pl.when(p == 0)
        def _conv_phase():
            @pl.when(g == 0)
            def _init():
                ss_ref[...] = jnp.zeros_like(ss_ref)
                sq_ref[...] = jnp.zeros_like(sq_ref)
                for i in range(NB):
                    xs_ref[i, :, :PADF] = jnp.zeros((Cin, PADF), jnp.bfloat16)
                    xs_ref[i, :, PADF + S:] = jnp.zeros(
                        (Cin, Lin - PADF - S), jnp.bfloat16)

            for i in range(NB):
                xs_ref[i, :, PADF:PADF + S] = x_ref[i].astype(jnp.bfloat16)
            for j, off in enumerate(offs):
                start = PADF - HW + off
                m = mask_ref[j:j + 1, :]
                for i in range(NB):
                    col_ref[i, j * Cin:(j + 1) * Cin, :] = (
                        xs_ref[i, :, start:start + Lc] * m)
            ps = None
            pq = None
            for i in range(NB):
                acc = (jnp.dot(w_ref[0], col_ref[i, :, 0:S],
                               preferred_element_type=jnp.float32)
                       + jnp.dot(w_ref[1], col_ref[i, :, HW:HW + S],
                                 preferred_element_type=jnp.float32)
                       + jnp.dot(w_ref[2], col_ref[i, :, 2 * HW:2 * HW + S],
                                 preferred_element_type=jnp.float32))
                y_ref[g * NB + i] = acc.astype(jnp.bfloat16)
                s = jnp.sum(acc, axis=1, keepdims=True)
                t = jnp.sum(acc * acc, axis=1, keepdims=True)
                ps = s if ps is None else ps + s
                pq = t if pq is None else pq + t
            ss_ref[...] += ps
            sq_ref[...] += pq

            @pl.when(g == G - 1)
            def _finalize_affine():
                mean = ss_ref[...] / count
                var = sq_ref[...] / count - mean * mean
                inv = gb_ref[0] * lax.rsqrt(var + _EPS)
                sc_ref[0] = inv
                sc_ref[1] = gb_ref[1] - mean * inv

        @pl.when(p == 1)
        def _bn_phase():
            sc = sc_ref[0]
            sh = sc_ref[1]
            for i in range(NB):
                o_ref[i] = jnp.maximum(
                    y_ref[g * NB + i].astype(jnp.float32) * sc + sh, 0.0)

    out = pl.pallas_call(
        fused_kernel,
        out_shape=jax.ShapeDtypeStruct((N, Cout, S), jnp.float32),
        grid_spec=pltpu.PrefetchScalarGridSpec(
            num_scalar_prefetch=0,
            grid=(2, G),
            in_specs=[
                pl.BlockSpec((NB, Cin, S), lambda p, g: ((1 - p) * g, 0, 0)),
                pl.BlockSpec((3, Cout, K9), lambda p, g: (0, 0, 0)),
                pl.BlockSpec((9, Lc), lambda p, g: (0, 0)),
                pl.BlockSpec((2, Cout, 1), lambda p, g: (0, 0, 0)),
            ],
            out_specs=pl.BlockSpec((NB, Cout, S), lambda p, g: (p * g, 0, 0)),
            scratch_shapes=[
                pltpu.VMEM((NB, Cin, Lin), jnp.bfloat16),
                pltpu.VMEM((NB, K9, Lc), jnp.bfloat16),
                pltpu.VMEM((N, Cout, S), jnp.bfloat16),
                pltpu.VMEM((Cout, 1), jnp.float32),
                pltpu.VMEM((Cout, 1), jnp.float32),
                pltpu.VMEM((2, Cout, 1), jnp.float32),
            ],
        ),
        compiler_params=pltpu.CompilerParams(
            dimension_semantics=("arbitrary", "arbitrary"),
            vmem_limit_bytes=64 * 1024 * 1024,
        ),
    )(x3, w3, mask_arr, gb)

    return out.reshape(N, Cout, D, H, W)


def kernel(x_ncdhw, w_oidhw, gamma, beta):
    return _conv3d_bn_relu(x_ncdhw, w_oidhw, gamma, beta)
